# baseline (speedup 1.0000x reference)
import functools

import jax
import jax.numpy as jnp
from jax.experimental import pallas as pl
from jax.experimental.pallas import tpu as pltpu

_BM = 2048


def _fused_body(x_hbm, w_ref, out_hbm, x_vmem, wp_ref, obuf, in_sems, out_sems):
    nb = x_vmem.shape[0] // _BM

    def _in_copy(k):
        return pltpu.make_async_copy(
            x_hbm.at[pl.ds(k * _BM, _BM), :],
            x_vmem.at[pl.ds(k * _BM, _BM), :],
            in_sems.at[k],
        )

    def _out_copy(j, slot):
        return pltpu.make_async_copy(
            obuf.at[slot],
            out_hbm.at[pl.ds(j * _BM, _BM), :],
            out_sems.at[slot],
        )

    for k in range(nb):
        _in_copy(k).start()

    acc = jnp.zeros((1, x_vmem.shape[1]), jnp.float32)
    for k in range(nb):
        _in_copy(k).wait()
        blk = x_vmem[pl.ds(k * _BM, _BM), :]
        acc = acc + jnp.sum(blk * blk, axis=0, keepdims=True)

    w = w_ref[...]
    cinv = jax.lax.rsqrt(acc)
    rinv = jax.lax.rsqrt(jnp.sum(w * w, axis=1, keepdims=True))
    wp = (w * cinv * rinv).astype(jnp.bfloat16)
    wp_ref[...] = wp

    for j in range(nb):
        slot = j % 2
        xblk = x_vmem[pl.ds(j * _BM, _BM), :].astype(jnp.bfloat16)
        beta = jax.lax.dot_general(
            xblk, wp,
            dimension_numbers=(((1,), (1,)), ((), ())),
            preferred_element_type=jnp.float32,
        )
        e = jnp.exp(beta)
        if j >= 2:
            _out_copy(j - 2, slot).wait()
        obuf[slot] = e * (1.0 / jnp.sum(e, axis=1, keepdims=True))
        _out_copy(j, slot).start()
    for j in (nb - 2, nb - 1):
        _out_copy(j, j % 2).wait()


@functools.partial(jax.jit, static_argnames=("interpret",))
def _snn_softmax(all_ts, W, interpret=False):
    B, TS = all_ts.shape
    N = W.shape[0]
    nb = B // _BM
    out = pl.pallas_call(
        _fused_body,
        grid=(1,),
        in_specs=[
            pl.BlockSpec(memory_space=pl.ANY),
            pl.BlockSpec((N, TS), lambda i: (0, 0)),
        ],
        out_specs=pl.BlockSpec(memory_space=pl.ANY),
        out_shape=jax.ShapeDtypeStruct((B, N), jnp.float32),
        scratch_shapes=[
            pltpu.VMEM((B, TS), jnp.float32),
            pltpu.VMEM((N, TS), jnp.bfloat16),
            pltpu.VMEM((2, _BM, N), jnp.float32),
            pltpu.SemaphoreType.DMA((nb,)),
            pltpu.SemaphoreType.DMA((2,)),
        ],
        interpret=interpret,
    )(all_ts, W)
    return out


def kernel(all_ts, W, cumhisto, clustering_flag):
    x = all_ts.reshape(all_ts.shape[0], -1)
    return _snn_softmax(x, W)
